# baseline (speedup 1.0000x reference)
import functools

import jax
import jax.numpy as jnp
from jax import lax
from jax.experimental import pallas as pl
from jax.experimental.pallas import tpu as pltpu
from jax.experimental.pallas import tpu_sc as plsc

_NC = 2
_NS = 16
_NW = _NC * _NS
_L = 16

_N = 10000
_E = 320000
_CH = 80
_EPW = _E // _NW
_NCHUNK = _EPW // _CH

_ZBIG = 640
_ZLAST = _N - 15 * _ZBIG

_PK = 16384


def _unpack_edges(pk_v, src_v, dst_v, need_src):
    def row(c, _):
        for g in range(_CH // _L):
            pv = pk_v[c, pl.ds(g * _L, _L)]
            if need_src:
                src_v[c, pl.ds(g * _L, _L)] = jnp.right_shift(pv, 14)
            dst_v[c, pl.ds(g * _L, _L)] = jnp.bitwise_and(pv, _PK - 1)
        return 0
    lax.fori_loop(0, _NCHUNK, row, 0)


def _zero_fill(buf, nvec):
    def body(i, _):
        buf[pl.ds(i * _L, _L)] = jnp.zeros((_L,), jnp.float32)
        return 0
    lax.fori_loop(0, nvec, body, 0)


def _deg_body(dst_hbm, w_hbm, out_hbm, dst_v, w_v, acc, zbuf):
    cid = lax.axis_index("c")
    sid = lax.axis_index("s")
    wid = sid * _NC + cid

    _zero_fill(zbuf, _ZBIG // _L)

    @pl.when(sid < 15)
    def _():
        pltpu.sync_copy(zbuf, acc.at[pl.ds(sid * _ZBIG, _ZBIG)])

    @pl.when(sid == 15)
    def _():
        pltpu.sync_copy(zbuf.at[pl.ds(0, _ZLAST)], acc.at[pl.ds(15 * _ZBIG, _ZLAST)])

    pltpu.sync_copy(dst_hbm.at[wid], dst_v)
    pltpu.sync_copy(w_hbm.at[wid], w_v)

    plsc.subcore_barrier()

    def chunk(c, _):
        pltpu.sync_copy(w_v.at[c], acc.at[dst_v.at[c]], add=True)
        return 0
    lax.fori_loop(0, _NCHUNK, chunk, 0)

    plsc.subcore_barrier()

    @pl.when(sid == 0)
    def _():
        pltpu.sync_copy(acc, out_hbm.at[cid])


def _sc_deg(dst3d, w3d):
    mesh = plsc.VectorSubcoreMesh(core_axis_name="c", subcore_axis_name="s")
    return pl.kernel(
        _deg_body,
        out_type=jax.ShapeDtypeStruct((_NC, _N), jnp.float32),
        mesh=mesh,
        scratch_types=[
            pltpu.VMEM((_NCHUNK, _CH), jnp.int32),
            pltpu.VMEM((_NCHUNK, _CH), jnp.float32),
            pltpu.VMEM_SHARED((_N,), jnp.float32),
            pltpu.VMEM((_ZBIG,), jnp.float32),
        ],
    )(dst3d, w3d)


def _agg_body(nj, pk_hbm, w_hbm, xs_hbm, out_hbm,
              pk_v, src_v, dst_v, w_v, rows0, rows1, acc,
              gsem0, gsem1, esem0, esem1, ssem0, ssem1, ssem2, ssem3):
    cid = lax.axis_index("c")
    sid = lax.axis_index("s")
    wid = sid * _NC + cid
    rows_b = (rows0, rows1)
    gsem_b = (gsem0, gsem1)
    esem_b = (esem0, esem1)
    ssem_q = (ssem0, ssem1, ssem2, ssem3)

    def zfill(e, _):
        for j in range(nj):
            rows0[e, pl.ds(j * _L, _L)] = jnp.zeros((_L,), jnp.float32)
        return 0
    lax.fori_loop(0, _CH, zfill, 0)

    @pl.when(sid < 15)
    def _():
        for k in range(_ZBIG // _CH):
            pltpu.sync_copy(rows0, acc.at[pl.ds(sid * _ZBIG + k * _CH, _CH)])

    @pl.when(sid == 15)
    def _():
        for k in range(_ZLAST // _CH):
            pltpu.sync_copy(rows0, acc.at[pl.ds(15 * _ZBIG + k * _CH, _CH)])

    plsc.subcore_barrier()

    def unpack(b, slot, c):
        for g in range(_CH // _L):
            pv = pk_v[b, pl.ds(g * _L, _L)]
            src_v[b, pl.ds(g * _L, _L)] = jnp.right_shift(pv, 14)
            dst_v[slot, pl.ds(g * _L, _L)] = jnp.bitwise_and(pv, _PK - 1)

    def escale(b, ws):
        rows = rows_b[b]

        def grp(g, _):
            wv = w_v[ws, pl.ds(g * _L, _L)]
            for l in range(_L):
                e = g * _L + l
                s = wv[l]
                for j in range(nj):
                    rows[e, pl.ds(j * _L, _L)] = rows[e, pl.ds(j * _L, _L)] * s
            return 0
        lax.fori_loop(0, _CH // _L, grp, 0)

    def wait_scatter(b, slot):
        pltpu.make_async_copy(rows_b[b], acc.at[dst_v.at[slot]],
                              ssem_q[slot]).wait()

    NQ = (_NCHUNK - 1) // 4

    pltpu.sync_copy(pk_hbm.at[wid].at[0], pk_v.at[0])
    pltpu.sync_copy(w_hbm.at[wid].at[0], w_v.at[0])
    pltpu.sync_copy(pk_hbm.at[wid].at[1], pk_v.at[1])
    pltpu.sync_copy(w_hbm.at[wid].at[1], w_v.at[1])
    unpack(0, 0, 0)
    unpack(1, 1, 1)
    pltpu.async_copy(xs_hbm.at[src_v.at[0]], rows0, gsem0)
    pltpu.async_copy(pk_hbm.at[wid].at[2], pk_v.at[0], esem0)
    pltpu.async_copy(w_hbm.at[wid].at[2], w_v.at[2], esem0)

    def quad(c4, _):
        for q in range(4):
            b = q % 2
            c = 4 * c4 + q
            pltpu.make_async_copy(xs_hbm.at[src_v.at[b]], rows_b[b],
                                  gsem_b[b]).wait()
            if q == 0:
                @pl.when(c4 > 0)
                def _():
                    wait_scatter(1 - b, 3)
            else:
                wait_scatter(1 - b, q - 1)
            pltpu.async_copy(xs_hbm.at[src_v.at[1 - b]], rows_b[1 - b],
                             gsem_b[1 - b])

            def edge_stage():
                pltpu.make_async_copy(pk_hbm.at[wid].at[c + 2], pk_v.at[b],
                                      esem_b[b]).wait()
                pltpu.make_async_copy(w_hbm.at[wid].at[c + 2],
                                      w_v.at[(q + 2) % 4], esem_b[b]).wait()
                unpack(b, (q + 2) % 4, c + 2)

            def edge_prefetch():
                pltpu.async_copy(pk_hbm.at[wid].at[c + 3], pk_v.at[1 - b],
                                 esem_b[1 - b])
                pltpu.async_copy(w_hbm.at[wid].at[c + 3],
                                 w_v.at[(q + 3) % 4], esem_b[1 - b])

            if q < 3:
                edge_stage()
            else:
                @pl.when(c4 < NQ - 1)
                def _():
                    edge_stage()
            if q < 2:
                edge_prefetch()
            else:
                @pl.when(c4 < NQ - 1)
                def _():
                    edge_prefetch()

            escale(b, q)
            pltpu.async_copy(rows_b[b], acc.at[dst_v.at[q]], ssem_q[q],
                             add=True)
        return 0
    lax.fori_loop(0, NQ, quad, 0)

    pltpu.make_async_copy(xs_hbm.at[src_v.at[0]], rows0, gsem0).wait()
    escale(0, 0)
    wait_scatter(1, 3)
    pltpu.sync_copy(rows0, acc.at[dst_v.at[0]], add=True)

    plsc.subcore_barrier()

    @pl.when(sid < 15)
    def _():
        pltpu.sync_copy(acc.at[pl.ds(sid * _ZBIG, _ZBIG)],
                        out_hbm.at[cid].at[pl.ds(sid * _ZBIG, _ZBIG)])

    @pl.when(sid == 15)
    def _():
        pltpu.sync_copy(acc.at[pl.ds(15 * _ZBIG, _ZLAST)],
                        out_hbm.at[cid].at[pl.ds(15 * _ZBIG, _ZLAST)])


def _sc_agg(pk3d, w3d, xs):
    d = xs.shape[1]
    mesh = plsc.VectorSubcoreMesh(core_axis_name="c", subcore_axis_name="s")
    params = None
    if d < 128:
        params = pltpu.CompilerParams(use_tc_tiling_on_sc=False)
    return pl.kernel(
        functools.partial(_agg_body, d // _L),
        out_type=jax.ShapeDtypeStruct((_NC, _N, d), jnp.float32),
        mesh=mesh,
        compiler_params=params,
        scratch_types=[
            pltpu.VMEM((2, _CH), jnp.int32),
            pltpu.VMEM((2, _CH), jnp.int32),
            pltpu.VMEM((4, _CH), jnp.int32),
            pltpu.VMEM((4, _CH), jnp.float32),
            pltpu.VMEM((_CH, d), jnp.float32),
            pltpu.VMEM((_CH, d), jnp.float32),
            pltpu.VMEM_SHARED((_N, d), jnp.float32),
            pltpu.SemaphoreType.DMA,
            pltpu.SemaphoreType.DMA,
            pltpu.SemaphoreType.DMA,
            pltpu.SemaphoreType.DMA,
            pltpu.SemaphoreType.DMA,
            pltpu.SemaphoreType.DMA,
            pltpu.SemaphoreType.DMA,
            pltpu.SemaphoreType.DMA,
        ],
    )(pk3d, w3d, xs)


_BLK = 1000


def _pack_body(s_ref, d_ref, p_ref):
    p_ref[...] = s_ref[...] * _PK + d_ref[...]


def _tc_pack(src, dst):
    s2 = src.reshape(_E // _BLK, _BLK)
    d2 = dst.reshape(_E // _BLK, _BLK)
    return pl.pallas_call(
        _pack_body,
        out_shape=jax.ShapeDtypeStruct((_E // _BLK, _BLK), jnp.int32),
    )(s2, d2)


def _xf1_body(d0_ref, d1_ref, x_ref, w_ref, xs_ref, dis_ref):
    deg = d0_ref[...] + d1_ref[...] + 1.0
    dis = lax.rsqrt(deg)
    dis_ref[...] = dis
    xt = jnp.dot(x_ref[...], w_ref[...], preferred_element_type=jnp.float32)
    xs_ref[...] = dis * xt


def _tc_xf1(d0, d1, x, W1):
    n, din = x.shape
    h1 = W1.shape[1]
    return pl.pallas_call(
        _xf1_body,
        grid=(n // _BLK,),
        in_specs=[
            pl.BlockSpec((_BLK, 1), lambda i: (i, 0)),
            pl.BlockSpec((_BLK, 1), lambda i: (i, 0)),
            pl.BlockSpec((_BLK, din), lambda i: (i, 0)),
            pl.BlockSpec((din, h1), lambda i: (0, 0)),
        ],
        out_specs=[
            pl.BlockSpec((_BLK, h1), lambda i: (i, 0)),
            pl.BlockSpec((_BLK, 1), lambda i: (i, 0)),
        ],
        out_shape=[
            jax.ShapeDtypeStruct((n, h1), jnp.float32),
            jax.ShapeDtypeStruct((n, 1), jnp.float32),
        ],
    )(d0, d1, x, W1)


def _xf2_body(s_ref, xs_ref, dis_ref, b_ref, w_ref, o_ref):
    dis = dis_ref[...]
    h = dis * (s_ref[0] + s_ref[1] + xs_ref[...]) + b_ref[...]
    h = jnp.maximum(h, 0.0)
    xt = jnp.dot(h, w_ref[...], preferred_element_type=jnp.float32)
    o_ref[...] = dis * xt


def _tc_xf2(S, xs, dis, b, W2):
    n, h1 = xs.shape
    h2 = W2.shape[1]
    return pl.pallas_call(
        _xf2_body,
        grid=(n // _BLK,),
        in_specs=[
            pl.BlockSpec((2, _BLK, h1), lambda i: (0, i, 0)),
            pl.BlockSpec((_BLK, h1), lambda i: (i, 0)),
            pl.BlockSpec((_BLK, 1), lambda i: (i, 0)),
            pl.BlockSpec((1, h1), lambda i: (0, 0)),
            pl.BlockSpec((h1, h2), lambda i: (0, 0)),
        ],
        out_specs=pl.BlockSpec((_BLK, h2), lambda i: (i, 0)),
        out_shape=jax.ShapeDtypeStruct((n, h2), jnp.float32),
    )(S, xs, dis, b.reshape(1, h1), W2)


def _head_body(s_ref, xs_ref, dis_ref, b_ref, wm_ref, bm_ref, o_ref):
    h = dis_ref[...] * (s_ref[0] + s_ref[1] + xs_ref[...]) + b_ref[...]
    h = jnp.maximum(h, 0.0)
    s = jnp.dot(h, wm_ref[...], preferred_element_type=jnp.float32) + bm_ref[...]
    s = s - jnp.max(s, axis=-1, keepdims=True)
    e = jnp.exp(s)
    o_ref[...] = e / jnp.sum(e, axis=-1, keepdims=True)


def _tc_head(S, xs, dis, b, Wm, bm):
    n = xs.shape[0]
    h2, k = Wm.shape
    return pl.pallas_call(
        _head_body,
        grid=(n // _BLK,),
        in_specs=[
            pl.BlockSpec((2, _BLK, h2), lambda i: (0, i, 0)),
            pl.BlockSpec((_BLK, h2), lambda i: (i, 0)),
            pl.BlockSpec((_BLK, 1), lambda i: (i, 0)),
            pl.BlockSpec((1, h2), lambda i: (0, 0)),
            pl.BlockSpec((h2, k), lambda i: (0, 0)),
            pl.BlockSpec((1, k), lambda i: (0, 0)),
        ],
        out_specs=pl.BlockSpec((_BLK, k), lambda i: (i, 0)),
        out_shape=jax.ShapeDtypeStruct((n, k), jnp.float32),
    )(S, xs, dis, b.reshape(1, h2), Wm, bm.reshape(1, k))


def kernel(x, edge_index, edge_weight, W1, b1, W2, b2, Wm, bm):
    pk3d = _tc_pack(edge_index[0], edge_index[1]).reshape(_NW, _NCHUNK, _CH)
    w3d = edge_weight.reshape(_NW, _NCHUNK, _CH)
    dst3d = edge_index[1].reshape(_NW, _NCHUNK, _CH)

    degp = _sc_deg(dst3d, w3d)
    d0 = degp[0].reshape(_N, 1)
    d1 = degp[1].reshape(_N, 1)

    xs1, dis = _tc_xf1(d0, d1, x, W1)
    S1 = _sc_agg(pk3d, w3d, xs1)
    xs2 = _tc_xf2(S1, xs1, dis, b1, W2)
    S2 = _sc_agg(pk3d, w3d, xs2)
    return _tc_head(S2, xs2, dis, b2, Wm, bm)


# speedup vs baseline: 1.1561x; 1.1561x over previous
import functools

import jax
import jax.numpy as jnp
from jax import lax
from jax.experimental import pallas as pl
from jax.experimental.pallas import tpu as pltpu
from jax.experimental.pallas import tpu_sc as plsc

_NC = 2
_NS = 16
_NW = _NC * _NS
_L = 16

_N = 10000
_E = 320000
_CH = 80
_EPW = _E // _NW
_NCHUNK = _EPW // _CH

_ZBIG = 640
_ZLAST = _N - 15 * _ZBIG

_PK = 16384


def _unpack_edges(pk_v, src_v, dst_v, need_src):
    def row(c, _):
        for g in range(_CH // _L):
            pv = pk_v[c, pl.ds(g * _L, _L)]
            if need_src:
                src_v[c, pl.ds(g * _L, _L)] = jnp.right_shift(pv, 14)
            dst_v[c, pl.ds(g * _L, _L)] = jnp.bitwise_and(pv, _PK - 1)
        return 0
    lax.fori_loop(0, _NCHUNK, row, 0)


def _zero_fill(buf, nvec):
    def body(i, _):
        buf[pl.ds(i * _L, _L)] = jnp.zeros((_L,), jnp.float32)
        return 0
    lax.fori_loop(0, nvec, body, 0)


def _deg_body(dst_hbm, w_hbm, out_hbm, dst_v, w_v, acc, zbuf):
    cid = lax.axis_index("c")
    sid = lax.axis_index("s")
    wid = sid * _NC + cid

    _zero_fill(zbuf, _ZBIG // _L)

    @pl.when(sid < 15)
    def _():
        pltpu.sync_copy(zbuf, acc.at[pl.ds(sid * _ZBIG, _ZBIG)])

    @pl.when(sid == 15)
    def _():
        pltpu.sync_copy(zbuf.at[pl.ds(0, _ZLAST)], acc.at[pl.ds(15 * _ZBIG, _ZLAST)])

    pltpu.sync_copy(dst_hbm.at[wid], dst_v)
    pltpu.sync_copy(w_hbm.at[wid], w_v)

    plsc.subcore_barrier()

    def chunk(c, _):
        pltpu.sync_copy(w_v.at[c], acc.at[dst_v.at[c]], add=True)
        return 0
    lax.fori_loop(0, _NCHUNK, chunk, 0)

    plsc.subcore_barrier()

    @pl.when(sid == 0)
    def _():
        pltpu.sync_copy(acc, out_hbm.at[cid])


def _sc_deg(dst3d, w3d):
    mesh = plsc.VectorSubcoreMesh(core_axis_name="c", subcore_axis_name="s")
    return pl.kernel(
        _deg_body,
        out_type=jax.ShapeDtypeStruct((_NC, _N), jnp.float32),
        mesh=mesh,
        scratch_types=[
            pltpu.VMEM((_NCHUNK, _CH), jnp.int32),
            pltpu.VMEM((_NCHUNK, _CH), jnp.float32),
            pltpu.VMEM_SHARED((_N,), jnp.float32),
            pltpu.VMEM((_ZBIG,), jnp.float32),
        ],
    )(dst3d, w3d)


def _agg_body(nj, pk_hbm, w_hbm, xs_hbm, out_hbm,
              pk_v, src_v, dst_v, w_v, rows0, rows1, acc,
              gsem0, gsem1, esem0, esem1, ssem0, ssem1, ssem2, ssem3):
    cid = lax.axis_index("c")
    sid = lax.axis_index("s")
    wid = sid * _NC + cid
    rows_b = (rows0, rows1)
    gsem_b = (gsem0, gsem1)
    esem_b = (esem0, esem1)
    ssem_q = (ssem0, ssem1, ssem2, ssem3)

    def zfill(e, _):
        for j in range(nj):
            rows0[e, pl.ds(j * _L, _L)] = jnp.zeros((_L,), jnp.float32)
        return 0
    lax.fori_loop(0, _CH, zfill, 0)

    @pl.when(sid < 15)
    def _():
        for k in range(_ZBIG // _CH):
            pltpu.sync_copy(rows0, acc.at[pl.ds(sid * _ZBIG + k * _CH, _CH)])

    @pl.when(sid == 15)
    def _():
        for k in range(_ZLAST // _CH):
            pltpu.sync_copy(rows0, acc.at[pl.ds(15 * _ZBIG + k * _CH, _CH)])

    plsc.subcore_barrier()

    def unpack(b, slot, c):
        for g in range(_CH // _L):
            pv = pk_v[b, pl.ds(g * _L, _L)]
            src_v[b, pl.ds(g * _L, _L)] = jnp.right_shift(pv, 14)
            dst_v[slot, pl.ds(g * _L, _L)] = jnp.bitwise_and(pv, _PK - 1)

    def escale(b, ws):
        rows = rows_b[b]

        def grp(g, _):
            wv = w_v[ws, pl.ds(g * _L, _L)]
            for l in range(_L):
                e = g * _L + l
                s = wv[l]
                for j in range(nj):
                    rows[e, pl.ds(j * _L, _L)] = rows[e, pl.ds(j * _L, _L)] * s
            return 0
        lax.fori_loop(0, _CH // _L, grp, 0)

    def wait_scatter(b, slot):
        pltpu.make_async_copy(rows_b[b], acc.at[dst_v.at[slot]],
                              ssem_q[slot]).wait()

    NQ = (_NCHUNK - 1) // 4

    pltpu.sync_copy(pk_hbm.at[wid].at[0], pk_v.at[0])
    pltpu.sync_copy(w_hbm.at[wid].at[0], w_v.at[0])
    pltpu.sync_copy(pk_hbm.at[wid].at[1], pk_v.at[1])
    pltpu.sync_copy(w_hbm.at[wid].at[1], w_v.at[1])
    unpack(0, 0, 0)
    unpack(1, 1, 1)
    pltpu.async_copy(xs_hbm.at[src_v.at[0]], rows0, gsem0)
    pltpu.async_copy(pk_hbm.at[wid].at[2], pk_v.at[0], esem0)
    pltpu.async_copy(w_hbm.at[wid].at[2], w_v.at[2], esem0)

    def quad(c4, _):
        for q in range(4):
            b = q % 2
            c = 4 * c4 + q
            pltpu.make_async_copy(xs_hbm.at[src_v.at[b]], rows_b[b],
                                  gsem_b[b]).wait()
            if q == 0:
                @pl.when(c4 > 0)
                def _():
                    wait_scatter(1 - b, 3)
            else:
                wait_scatter(1 - b, q - 1)
            pltpu.async_copy(xs_hbm.at[src_v.at[1 - b]], rows_b[1 - b],
                             gsem_b[1 - b])

            def edge_stage():
                pltpu.make_async_copy(pk_hbm.at[wid].at[c + 2], pk_v.at[b],
                                      esem_b[b]).wait()
                pltpu.make_async_copy(w_hbm.at[wid].at[c + 2],
                                      w_v.at[(q + 2) % 4], esem_b[b]).wait()
                unpack(b, (q + 2) % 4, c + 2)

            def edge_prefetch():
                pltpu.async_copy(pk_hbm.at[wid].at[c + 3], pk_v.at[1 - b],
                                 esem_b[1 - b])
                pltpu.async_copy(w_hbm.at[wid].at[c + 3],
                                 w_v.at[(q + 3) % 4], esem_b[1 - b])

            if q < 3:
                edge_stage()
            else:
                @pl.when(c4 < NQ - 1)
                def _():
                    edge_stage()
            if q < 2:
                edge_prefetch()
            else:
                @pl.when(c4 < NQ - 1)
                def _():
                    edge_prefetch()

            escale(b, q)
            pltpu.async_copy(rows_b[b], acc.at[dst_v.at[q]], ssem_q[q],
                             add=True)
        return 0
    lax.fori_loop(0, NQ, quad, 0)

    pltpu.make_async_copy(xs_hbm.at[src_v.at[0]], rows0, gsem0).wait()
    escale(0, 0)
    wait_scatter(1, 3)
    pltpu.sync_copy(rows0, acc.at[dst_v.at[0]], add=True)

    plsc.subcore_barrier()

    @pl.when(sid < 15)
    def _():
        pltpu.sync_copy(acc.at[pl.ds(sid * _ZBIG, _ZBIG)],
                        out_hbm.at[cid].at[pl.ds(sid * _ZBIG, _ZBIG)])

    @pl.when(sid == 15)
    def _():
        pltpu.sync_copy(acc.at[pl.ds(15 * _ZBIG, _ZLAST)],
                        out_hbm.at[cid].at[pl.ds(15 * _ZBIG, _ZLAST)])


def _sc_agg(pk3d, w3d, xs):
    d = xs.shape[1]
    mesh = plsc.VectorSubcoreMesh(core_axis_name="c", subcore_axis_name="s")
    params = None
    if d < 128:
        params = pltpu.CompilerParams(use_tc_tiling_on_sc=False)
    return pl.kernel(
        functools.partial(_agg_body, d // _L),
        out_type=jax.ShapeDtypeStruct((_NC, _N, d), jnp.float32),
        mesh=mesh,
        compiler_params=params,
        scratch_types=[
            pltpu.VMEM((2, _CH), jnp.int32),
            pltpu.VMEM((2, _CH), jnp.int32),
            pltpu.VMEM((4, _CH), jnp.int32),
            pltpu.VMEM((4, _CH), jnp.float32),
            pltpu.VMEM((_CH, d), jnp.float32),
            pltpu.VMEM((_CH, d), jnp.float32),
            pltpu.VMEM_SHARED((_N, d), jnp.float32),
            pltpu.SemaphoreType.DMA,
            pltpu.SemaphoreType.DMA,
            pltpu.SemaphoreType.DMA,
            pltpu.SemaphoreType.DMA,
            pltpu.SemaphoreType.DMA,
            pltpu.SemaphoreType.DMA,
            pltpu.SemaphoreType.DMA,
            pltpu.SemaphoreType.DMA,
        ],
    )(pk3d, w3d, xs)


_BLK = 1000


def _pack_body(s_ref, d_ref, p_ref):
    p_ref[...] = s_ref[...] * _PK + d_ref[...]


def _tc_pack(src, dst):
    s2 = src.reshape(_E // _BLK, _BLK)
    d2 = dst.reshape(_E // _BLK, _BLK)
    return pl.pallas_call(
        _pack_body,
        out_shape=jax.ShapeDtypeStruct((_E // _BLK, _BLK), jnp.int32),
    )(s2, d2)


def _xf1_body(d0_ref, d1_ref, x_ref, w_ref, xs_ref, dis_ref):
    deg = d0_ref[...] + d1_ref[...] + 1.0
    dis = lax.rsqrt(deg)
    dis_ref[...] = dis
    xt = jnp.dot(x_ref[...], w_ref[...], preferred_element_type=jnp.float32)
    xs_ref[...] = dis * xt


def _tc_xf1(d0, d1, x, W1):
    n, din = x.shape
    h1 = W1.shape[1]
    return pl.pallas_call(
        _xf1_body,
        grid=(n // _BLK,),
        in_specs=[
            pl.BlockSpec((_BLK, 1), lambda i: (i, 0)),
            pl.BlockSpec((_BLK, 1), lambda i: (i, 0)),
            pl.BlockSpec((_BLK, din), lambda i: (i, 0)),
            pl.BlockSpec((din, h1), lambda i: (0, 0)),
        ],
        out_specs=[
            pl.BlockSpec((_BLK, h1), lambda i: (i, 0)),
            pl.BlockSpec((_BLK, 1), lambda i: (i, 0)),
        ],
        out_shape=[
            jax.ShapeDtypeStruct((n, h1), jnp.float32),
            jax.ShapeDtypeStruct((n, 1), jnp.float32),
        ],
    )(d0, d1, x, W1)


def _xf2_body(s_ref, xs_ref, dis_ref, b_ref, w_ref, o_ref):
    dis = dis_ref[...]
    h = dis * (s_ref[0] + s_ref[1] + xs_ref[...]) + b_ref[...]
    h = jnp.maximum(h, 0.0)
    xt = jnp.dot(h, w_ref[...], preferred_element_type=jnp.float32)
    o_ref[...] = jnp.concatenate([dis * xt, jnp.zeros_like(xt)], axis=1)


def _tc_xf2(S, xs, dis, b, W2):
    n, h1 = xs.shape
    h2 = W2.shape[1]
    return pl.pallas_call(
        _xf2_body,
        grid=(n // _BLK,),
        in_specs=[
            pl.BlockSpec((2, _BLK, h1), lambda i: (0, i, 0)),
            pl.BlockSpec((_BLK, h1), lambda i: (i, 0)),
            pl.BlockSpec((_BLK, 1), lambda i: (i, 0)),
            pl.BlockSpec((1, h1), lambda i: (0, 0)),
            pl.BlockSpec((h1, h2), lambda i: (0, 0)),
        ],
        out_specs=pl.BlockSpec((_BLK, 2 * h2), lambda i: (i, 0)),
        out_shape=jax.ShapeDtypeStruct((n, 2 * h2), jnp.float32),
    )(S, xs, dis, b.reshape(1, h1), W2)


def _head_body(s_ref, xs_ref, dis_ref, b_ref, wm_ref, bm_ref, o_ref):
    acc = (s_ref[0] + s_ref[1] + xs_ref[...])[:, :64]
    h = dis_ref[...] * acc + b_ref[...]
    h = jnp.maximum(h, 0.0)
    s = jnp.dot(h, wm_ref[...], preferred_element_type=jnp.float32) + bm_ref[...]
    s = s - jnp.max(s, axis=-1, keepdims=True)
    e = jnp.exp(s)
    o_ref[...] = e / jnp.sum(e, axis=-1, keepdims=True)


def _tc_head(S, xs, dis, b, Wm, bm):
    n = xs.shape[0]
    h2, k = Wm.shape
    return pl.pallas_call(
        _head_body,
        grid=(n // _BLK,),
        in_specs=[
            pl.BlockSpec((2, _BLK, 128), lambda i: (0, i, 0)),
            pl.BlockSpec((_BLK, 128), lambda i: (i, 0)),
            pl.BlockSpec((_BLK, 1), lambda i: (i, 0)),
            pl.BlockSpec((1, h2), lambda i: (0, 0)),
            pl.BlockSpec((h2, k), lambda i: (0, 0)),
            pl.BlockSpec((1, k), lambda i: (0, 0)),
        ],
        out_specs=pl.BlockSpec((_BLK, k), lambda i: (i, 0)),
        out_shape=jax.ShapeDtypeStruct((n, k), jnp.float32),
    )(S, xs, dis, b.reshape(1, h2), Wm, bm.reshape(1, k))


def kernel(x, edge_index, edge_weight, W1, b1, W2, b2, Wm, bm):
    pk3d = _tc_pack(edge_index[0], edge_index[1]).reshape(_NW, _NCHUNK, _CH)
    w3d = edge_weight.reshape(_NW, _NCHUNK, _CH)
    dst3d = edge_index[1].reshape(_NW, _NCHUNK, _CH)

    degp = _sc_deg(dst3d, w3d)
    d0 = degp[0].reshape(_N, 1)
    d1 = degp[1].reshape(_N, 1)

    xs1, dis = _tc_xf1(d0, d1, x, W1)
    S1 = _sc_agg(pk3d, w3d, xs1)
    xs2 = _tc_xf2(S1, xs1, dis, b1, W2)
    S2 = _sc_agg(pk3d, w3d, xs2)
    return _tc_head(S2, xs2, dis, b2, Wm, bm)
